# baseline (speedup 1.0000x reference)
import functools

import jax
import jax.numpy as jnp
from jax import lax
from jax.experimental import pallas as pl
from jax.experimental.pallas import tpu as pltpu


def _fused(src_ref, w_src_ref, w_edge_ref, bias_ref, iota_s_ref, iota_d_ref,
           sid0_ref, did0_ref, edge_ref, sidb_ref, didb_ref, sidn_ref,
           didn_ref, out_ref, sproj_ref, acc_ref, ohs_a, ohd_a, ohs_b, ohd_b,
           *, nd, fo, ts, n_steps):
    step = pl.program_id(0)
    one = jnp.bfloat16(1.0)
    zero = jnp.bfloat16(0.0)

    def _gen(sid, did, ohs_ref, ohd_ref):
        ohs_ref[...] = jnp.where(iota_s_ref[...] == sid, one, zero)
        ohd_ref[...] = jnp.where(iota_d_ref[...] == did, one, zero)

    @pl.when(step == 0)
    def _():
        sp_t = lax.dot_general(w_src_ref[...],
                               src_ref[...].astype(jnp.bfloat16),
                               (((0,), (1,)), ((), ())),
                               preferred_element_type=jnp.float32)
        sproj_ref[...] = sp_t.astype(jnp.bfloat16)
        _gen(sid0_ref[...], did0_ref[...], ohs_a, ohd_a)

    def _compute(ohs_ref, ohd_ref, edge, init):
        gath_t = lax.dot_general(sproj_ref[...], ohs_ref[...],
                                 (((1,), (0,)), ((), ())),
                                 preferred_element_type=jnp.float32)
        ep_t = lax.dot_general(w_edge_ref[...], edge.astype(jnp.bfloat16),
                               (((0,), (1,)), ((), ())),
                               preferred_element_type=jnp.float32)
        msg_t = jnp.concatenate(
            [(gath_t + ep_t).astype(jnp.bfloat16),
             jnp.ones((8, ts), jnp.bfloat16)], axis=0)
        contrib = lax.dot_general(msg_t, ohd_ref[...],
                                  (((1,), (0,)), ((), ())),
                                  preferred_element_type=jnp.float32)
        if init:
            acc_ref[...] = jnp.where(step == 0, contrib,
                                     acc_ref[...] + contrib)
        else:
            acc_ref[...] += contrib

    _compute(ohs_a, ohd_a, edge_ref[0:ts, :], True)
    _gen(sidb_ref[...], didb_ref[...], ohs_b, ohd_b)
    _compute(ohs_b, ohd_b, edge_ref[ts:2 * ts, :], False)
    _gen(sidn_ref[...], didn_ref[...], ohs_a, ohd_a)

    @pl.when(step == n_steps - 1)
    def _():
        acc = acc_ref[0:fo, :]
        deg = acc_ref[fo:fo + 1, :]
        norm = jnp.where(deg > 0, lax.rsqrt(deg), 0.0)
        out_t = acc * norm + bias_ref[...]
        out_ref[...] = out_t.T


def kernel(src_feats, edge_feats, src_ids, dst_ids, weights, bias,
           n_dst=2048, ts=2048):
    f32 = jnp.float32
    bf16 = jnp.bfloat16
    n_src, in_feat = src_feats.shape
    n_edges = edge_feats.shape[0]
    out_feat = weights.shape[1]

    assert n_edges % (2 * ts) == 0
    n_tiles = n_edges // ts
    n_steps = n_tiles // 2

    w = weights.astype(f32)
    w_src = w[:in_feat].astype(bf16)
    w_edge = w[in_feat:].astype(bf16)

    def _pat(x):
        return lax.bitcast_convert_type(
            (x + 16384).astype(jnp.uint16), jnp.bfloat16)

    sid_row = _pat(src_ids.astype(jnp.int32).reshape(1, n_edges))
    did_col = _pat(dst_ids.astype(jnp.int32).reshape(n_edges, 1))
    iota_s = _pat(jnp.arange(n_src, dtype=jnp.int32).reshape(n_src, 1))
    iota_d = _pat(jnp.arange(n_dst, dtype=jnp.int32).reshape(1, n_dst))
    bias_col = bias.astype(f32).reshape(out_feat, 1)

    last = n_tiles - 1

    out = pl.pallas_call(
        functools.partial(_fused, nd=n_dst, fo=out_feat, ts=ts,
                          n_steps=n_steps),
        grid=(n_steps,),
        in_specs=[
            pl.BlockSpec((n_src, in_feat), lambda k: (0, 0)),
            pl.BlockSpec((in_feat, out_feat), lambda k: (0, 0)),
            pl.BlockSpec((in_feat, out_feat), lambda k: (0, 0)),
            pl.BlockSpec((out_feat, 1), lambda k: (0, 0)),
            pl.BlockSpec((n_src, 1), lambda k: (0, 0)),
            pl.BlockSpec((1, n_dst), lambda k: (0, 0)),
            pl.BlockSpec((1, ts), lambda k: (0, 0)),
            pl.BlockSpec((ts, 1), lambda k: (0, 0)),
            pl.BlockSpec((2 * ts, in_feat), lambda k: (k, 0)),
            pl.BlockSpec((1, ts), lambda k: (0, 2 * k + 1)),
            pl.BlockSpec((ts, 1), lambda k: (2 * k + 1, 0)),
            pl.BlockSpec((1, ts),
                         lambda k: (0, jnp.minimum(2 * k + 2, last))),
            pl.BlockSpec((ts, 1),
                         lambda k: (jnp.minimum(2 * k + 2, last), 0)),
        ],
        out_specs=pl.BlockSpec((n_dst, out_feat), lambda k: (0, 0)),
        out_shape=jax.ShapeDtypeStruct((n_dst, out_feat), f32),
        scratch_shapes=[
            pltpu.VMEM((out_feat, n_src), bf16),
            pltpu.VMEM((out_feat + 8, n_dst), f32),
            pltpu.VMEM((n_src, ts), bf16),
            pltpu.VMEM((ts, n_dst), bf16),
            pltpu.VMEM((n_src, ts), bf16),
            pltpu.VMEM((ts, n_dst), bf16),
        ],
        compiler_params=pltpu.CompilerParams(
            dimension_semantics=("arbitrary",),
            vmem_limit_bytes=100 * 1024 * 1024),
    )(src_feats, w_src, w_edge, bias_col, iota_s, iota_d,
      sid_row[:, :ts], did_col[:ts, :], edge_feats,
      sid_row, did_col, sid_row, did_col)

    return out


# speedup vs baseline: 1.4587x; 1.4587x over previous
import functools

import jax
import jax.numpy as jnp
from jax import lax
from jax.experimental import pallas as pl
from jax.experimental.pallas import tpu as pltpu


def _fused(src_ref, w_src_ref, w_edge_ref, bias_ref, edge_ref, sid_ref,
           did_ref, out_ref, sproj_ref, acc_ref, *, ns, nd, fo, te, n_tiles):
    step = pl.program_id(0)

    @pl.when(step == 0)
    def _():
        sp_t = lax.dot_general(w_src_ref[...],
                               src_ref[...].astype(jnp.bfloat16),
                               (((0,), (1,)), ((), ())),
                               preferred_element_type=jnp.float32)
        sproj_ref[...] = sp_t.astype(jnp.bfloat16)

    sid = sid_ref[...]
    oh_src = (lax.broadcasted_iota(jnp.int32, (ns, te), 0)
              == sid).astype(jnp.bfloat16)
    gath_t = lax.dot_general(sproj_ref[...], oh_src,
                             (((1,), (0,)), ((), ())),
                             preferred_element_type=jnp.float32)

    ep_t = lax.dot_general(w_edge_ref[...],
                           edge_ref[...].astype(jnp.bfloat16),
                           (((0,), (1,)), ((), ())),
                           preferred_element_type=jnp.float32)

    msg_t = jnp.concatenate(
        [(gath_t + ep_t).astype(jnp.bfloat16),
         jnp.ones((8, te), jnp.bfloat16)], axis=0)

    did = did_ref[...]
    oh_dst = (lax.broadcasted_iota(jnp.int32, (te, nd), 1)
              == did).astype(jnp.bfloat16)
    contrib = lax.dot_general(msg_t, oh_dst, (((1,), (0,)), ((), ())),
                              preferred_element_type=jnp.float32)

    @pl.when(step == 0)
    def _():
        acc_ref[...] = contrib

    @pl.when(step != 0)
    def _():
        acc_ref[...] += contrib

    @pl.when(step == n_tiles - 1)
    def _():
        acc = acc_ref[0:fo, :]
        deg = acc_ref[fo:fo + 1, :]
        norm = jnp.where(deg > 0, lax.rsqrt(deg), 0.0)
        out_t = acc * norm + bias_ref[...]
        out_ref[...] = out_t.T


def kernel(src_feats, edge_feats, src_ids, dst_ids, weights, bias,
           n_dst=2048, te=4096):
    f32 = jnp.float32
    bf16 = jnp.bfloat16
    n_src, in_feat = src_feats.shape
    n_edges = edge_feats.shape[0]
    out_feat = weights.shape[1]

    assert n_edges % te == 0
    n_tiles = n_edges // te
    m = out_feat + 8

    w = weights.astype(f32)
    w_src = w[:in_feat].astype(bf16)
    w_edge = w[in_feat:].astype(bf16)

    sid_row = src_ids.astype(jnp.int32).reshape(1, n_edges)
    did_col = dst_ids.astype(jnp.int32).reshape(n_edges, 1)
    bias_col = bias.astype(f32).reshape(out_feat, 1)

    out = pl.pallas_call(
        functools.partial(_fused, ns=n_src, nd=n_dst, fo=out_feat, te=te,
                          n_tiles=n_tiles),
        grid=(n_tiles,),
        in_specs=[
            pl.BlockSpec((n_src, in_feat), lambda e: (0, 0)),
            pl.BlockSpec((in_feat, out_feat), lambda e: (0, 0)),
            pl.BlockSpec((in_feat, out_feat), lambda e: (0, 0)),
            pl.BlockSpec((out_feat, 1), lambda e: (0, 0)),
            pl.BlockSpec((te, in_feat), lambda e: (e, 0)),
            pl.BlockSpec((1, te), lambda e: (0, e)),
            pl.BlockSpec((te, 1), lambda e: (e, 0)),
        ],
        out_specs=pl.BlockSpec((n_dst, out_feat), lambda e: (0, 0)),
        out_shape=jax.ShapeDtypeStruct((n_dst, out_feat), f32),
        scratch_shapes=[
            pltpu.VMEM((out_feat, n_src), bf16),
            pltpu.VMEM((m, n_dst), f32),
        ],
        compiler_params=pltpu.CompilerParams(
            dimension_semantics=("arbitrary",),
            vmem_limit_bytes=100 * 1024 * 1024),
    )(src_feats, w_src, w_edge, bias_col, edge_feats, sid_row, did_col)

    return out
